# baseline (speedup 1.0000x reference)
import functools

import jax
import jax.numpy as jnp
from jax.experimental import pallas as pl
from jax.experimental.pallas import tpu as pltpu

_NB = 4
_K = 1024
_D = 128
_ROWS = 8 * 576
_TILE = 512


def _vq_body(z_ref, cb_ref, codes_ref, inds_ref, comm_ref, c2_ref):
    j = pl.program_id(1)
    z = z_ref[...]
    cb = cb_ref[0]

    @pl.when(j == 0)
    def _c2():
        c2_ref[0, :] = jnp.sum(cb * cb, axis=1)

    dots = jnp.dot(z, cb.T, preferred_element_type=jnp.float32)
    z2 = jnp.sum(z * z, axis=1, keepdims=True)
    c2 = c2_ref[0, :]
    dist = z2 - 2.0 * dots + c2[None, :]
    m = jnp.min(dist, axis=1)
    lane = jax.lax.broadcasted_iota(jnp.int32, (_TILE, _K), 1)
    idx = jnp.min(jnp.where(dist == m[:, None], lane, _K), axis=1)
    onehot = (lane == idx[:, None]).astype(jnp.bfloat16)
    q = jnp.dot(onehot, cb.astype(jnp.bfloat16),
                preferred_element_type=jnp.float32)
    codes_ref[...] = q
    inds_ref[0, 0, :] = idx
    s = jnp.sum(m.reshape(_TILE // _D, _D), axis=0)

    @pl.when(j == 0)
    def _init():
        comm_ref[0, 0, :] = s

    @pl.when(j > 0)
    def _acc():
        comm_ref[0, 0, :] += s


@functools.partial(jax.jit)
def kernel(x, codebooks):
    b, n, D = x.shape
    xr = x.reshape(b * n, D)
    ntiles = _ROWS // _TILE
    codes, inds3, comm = pl.pallas_call(
        _vq_body,
        grid=(_NB, ntiles),
        in_specs=[
            pl.BlockSpec((_TILE, _D), lambda i, j: (j, i)),
            pl.BlockSpec((1, _K, _D), lambda i, j: (i, 0, 0)),
        ],
        out_specs=[
            pl.BlockSpec((_TILE, _D), lambda i, j: (j, i)),
            pl.BlockSpec((1, 1, _TILE), lambda i, j: (i, 0, j)),
            pl.BlockSpec((1, 1, _D), lambda i, j: (i, 0, 0)),
        ],
        out_shape=[
            jax.ShapeDtypeStruct((_ROWS, _NB * _D), jnp.float32),
            jax.ShapeDtypeStruct((_NB, 1, _ROWS), jnp.int32),
            jax.ShapeDtypeStruct((_NB, 1, _D), jnp.float32),
        ],
        scratch_shapes=[pltpu.VMEM((1, _K), jnp.float32)],
    )(xr, codebooks)
    codes = codes.reshape(b, n, D)
    inds = inds3.reshape(_NB, b, n).transpose(1, 2, 0)
    commits = jnp.sum(comm[:, 0, :], axis=-1) / jnp.float32(_ROWS * _D)
    return (codes, inds, commits)


# speedup vs baseline: 1.0536x; 1.0536x over previous
import functools

import jax
import jax.numpy as jnp
from jax.experimental import pallas as pl
from jax.experimental.pallas import tpu as pltpu

_NB = 4
_K = 1024
_D = 128
_ROWS = 8 * 576
_TILE = 512


def _vq_body(z_ref, cb_ref, c2_ref, codes_ref, inds_ref, comm_ref):
    j = pl.program_id(1)
    z = z_ref[...]
    cb = cb_ref[0]
    c2 = c2_ref[0, 0, :]
    dots = jnp.dot(z, cb.T, preferred_element_type=jnp.float32)
    z2 = jnp.sum(z * z, axis=1, keepdims=True)
    dist = z2 - 2.0 * dots + c2[None, :]
    m = jnp.min(dist, axis=1)
    lane_row = jax.lax.broadcasted_iota(jnp.int32, (1, _K), 1)
    lane_f = lane_row.astype(jnp.float32)
    idx_f = jnp.min(jnp.where(dist == m[:, None], lane_f,
                              jnp.float32(_K)), axis=1)
    idx = idx_f.astype(jnp.int32)
    onehot = (lane_row == idx[:, None]).astype(jnp.bfloat16)
    q = jnp.dot(onehot, cb.astype(jnp.bfloat16),
                preferred_element_type=jnp.float32)
    codes_ref[...] = q
    inds_ref[0, 0, :] = idx
    s = jnp.sum(m.reshape(_TILE // _D, _D), axis=0)

    @pl.when(j == 0)
    def _init():
        comm_ref[0, 0, :] = s

    @pl.when(j > 0)
    def _acc():
        comm_ref[0, 0, :] += s


@functools.partial(jax.jit)
def kernel(x, codebooks):
    b, n, D = x.shape
    xr = x.reshape(b * n, D)
    c2in = jnp.sum(codebooks * codebooks, axis=-1)[:, None, :]
    ntiles = _ROWS // _TILE
    codes, inds3, comm = pl.pallas_call(
        _vq_body,
        grid=(_NB, ntiles),
        in_specs=[
            pl.BlockSpec((_TILE, _D), lambda i, j: (j, i)),
            pl.BlockSpec((1, _K, _D), lambda i, j: (i, 0, 0)),
            pl.BlockSpec((1, 1, _K), lambda i, j: (i, 0, 0)),
        ],
        out_specs=[
            pl.BlockSpec((_TILE, _D), lambda i, j: (j, i)),
            pl.BlockSpec((1, 1, _TILE), lambda i, j: (i, 0, j)),
            pl.BlockSpec((1, 1, _D), lambda i, j: (i, 0, 0)),
        ],
        out_shape=[
            jax.ShapeDtypeStruct((_ROWS, _NB * _D), jnp.float32),
            jax.ShapeDtypeStruct((_NB, 1, _ROWS), jnp.int32),
            jax.ShapeDtypeStruct((_NB, 1, _D), jnp.float32),
        ],
    )(xr, codebooks, c2in)
    codes = codes.reshape(b, n, D)
    inds = inds3.reshape(_NB, b, n).transpose(1, 2, 0)
    commits = jnp.sum(comm[:, 0, :], axis=-1) / jnp.float32(_ROWS * _D)
    return (codes, inds, commits)


# speedup vs baseline: 1.1926x; 1.1319x over previous
import functools

import jax
import jax.numpy as jnp
from jax.experimental import pallas as pl
from jax.experimental.pallas import tpu as pltpu

_NB = 4
_K = 1024
_D = 128
_ROWS = 8 * 576
_TILE = 1152


def _vq_body(z_ref, cb_ref, c2_ref, codes_ref, inds_ref, comm_ref):
    j = pl.program_id(1)
    z = z_ref[...]
    cb = cb_ref[0]
    c2 = c2_ref[0, 0, :]
    dots = jnp.dot(z, cb.T, preferred_element_type=jnp.float32)
    z2 = jnp.sum(z * z, axis=1, keepdims=True)
    dist = z2 - 2.0 * dots + c2[None, :]
    m = jnp.min(dist, axis=1)
    lane_row = jax.lax.broadcasted_iota(jnp.int32, (1, _K), 1)
    lane_f = lane_row.astype(jnp.float32)
    idx_f = jnp.min(jnp.where(dist == m[:, None], lane_f,
                              jnp.float32(_K)), axis=1)
    idx = idx_f.astype(jnp.int32)
    onehot = (lane_row == idx[:, None]).astype(jnp.bfloat16)
    q = jnp.dot(onehot, cb.astype(jnp.bfloat16),
                preferred_element_type=jnp.float32)
    codes_ref[...] = q
    inds_ref[0, 0, :] = idx
    s = jnp.sum(m.reshape(_TILE // _D, _D), axis=0)

    @pl.when(j == 0)
    def _init():
        comm_ref[0, 0, :] = s

    @pl.when(j > 0)
    def _acc():
        comm_ref[0, 0, :] += s


@functools.partial(jax.jit)
def kernel(x, codebooks):
    b, n, D = x.shape
    xr = x.reshape(b * n, D)
    c2in = jnp.sum(codebooks * codebooks, axis=-1)[:, None, :]
    ntiles = _ROWS // _TILE
    codes, inds3, comm = pl.pallas_call(
        _vq_body,
        grid=(_NB, ntiles),
        in_specs=[
            pl.BlockSpec((_TILE, _D), lambda i, j: (j, i)),
            pl.BlockSpec((1, _K, _D), lambda i, j: (i, 0, 0)),
            pl.BlockSpec((1, 1, _K), lambda i, j: (i, 0, 0)),
        ],
        out_specs=[
            pl.BlockSpec((_TILE, _D), lambda i, j: (j, i)),
            pl.BlockSpec((1, 1, _TILE), lambda i, j: (i, 0, j)),
            pl.BlockSpec((1, 1, _D), lambda i, j: (i, 0, 0)),
        ],
        out_shape=[
            jax.ShapeDtypeStruct((_ROWS, _NB * _D), jnp.float32),
            jax.ShapeDtypeStruct((_NB, 1, _ROWS), jnp.int32),
            jax.ShapeDtypeStruct((_NB, 1, _D), jnp.float32),
        ],
    )(xr, codebooks, c2in)
    codes = codes.reshape(b, n, D)
    inds = inds3.reshape(_NB, b, n).transpose(1, 2, 0)
    commits = jnp.sum(comm[:, 0, :], axis=-1) / jnp.float32(_ROWS * _D)
    return (codes, inds, commits)


# speedup vs baseline: 1.2610x; 1.0574x over previous
import functools

import jax
import jax.numpy as jnp
from jax.experimental import pallas as pl
from jax.experimental.pallas import tpu as pltpu

_NB = 4
_K = 1024
_D = 128
_ROWS = 8 * 576
_TILE = 2304


def _vq_body(z_ref, cb_ref, c2_ref, codes_ref, inds_ref, comm_ref):
    j = pl.program_id(1)
    z = z_ref[...]
    cb = cb_ref[0]
    c2 = c2_ref[0, 0, :]
    dots = jnp.dot(z, cb.T, preferred_element_type=jnp.float32)
    z2 = jnp.sum(z * z, axis=1, keepdims=True)
    dist = z2 - 2.0 * dots + c2[None, :]
    m = jnp.min(dist, axis=1)
    lane_row = jax.lax.broadcasted_iota(jnp.int32, (1, _K), 1)
    lane_f = lane_row.astype(jnp.float32)
    idx_f = jnp.min(jnp.where(dist == m[:, None], lane_f,
                              jnp.float32(_K)), axis=1)
    idx = idx_f.astype(jnp.int32)
    onehot = (lane_row == idx[:, None]).astype(jnp.bfloat16)
    q = jnp.dot(onehot, cb.astype(jnp.bfloat16),
                preferred_element_type=jnp.float32)
    codes_ref[...] = q
    inds_ref[0, 0, :] = idx
    s = jnp.sum(m.reshape(_TILE // _D, _D), axis=0)

    @pl.when(j == 0)
    def _init():
        comm_ref[0, 0, :] = s

    @pl.when(j > 0)
    def _acc():
        comm_ref[0, 0, :] += s


@functools.partial(jax.jit)
def kernel(x, codebooks):
    b, n, D = x.shape
    xr = x.reshape(b * n, D)
    c2in = jnp.sum(codebooks * codebooks, axis=-1)[:, None, :]
    ntiles = _ROWS // _TILE
    codes, inds3, comm = pl.pallas_call(
        _vq_body,
        grid=(_NB, ntiles),
        in_specs=[
            pl.BlockSpec((_TILE, _D), lambda i, j: (j, i)),
            pl.BlockSpec((1, _K, _D), lambda i, j: (i, 0, 0)),
            pl.BlockSpec((1, 1, _K), lambda i, j: (i, 0, 0)),
        ],
        out_specs=[
            pl.BlockSpec((_TILE, _D), lambda i, j: (j, i)),
            pl.BlockSpec((1, 1, _TILE), lambda i, j: (i, 0, j)),
            pl.BlockSpec((1, 1, _D), lambda i, j: (i, 0, 0)),
        ],
        out_shape=[
            jax.ShapeDtypeStruct((_ROWS, _NB * _D), jnp.float32),
            jax.ShapeDtypeStruct((_NB, 1, _ROWS), jnp.int32),
            jax.ShapeDtypeStruct((_NB, 1, _D), jnp.float32),
        ],
    )(xr, codebooks, c2in)
    codes = codes.reshape(b, n, D)
    inds = inds3.reshape(_NB, b, n).transpose(1, 2, 0)
    commits = jnp.sum(comm[:, 0, :], axis=-1) / jnp.float32(_ROWS * _D)
    return (codes, inds, commits)


# speedup vs baseline: 1.3091x; 1.0381x over previous
import functools

import jax
import jax.numpy as jnp
from jax.experimental import pallas as pl
from jax.experimental.pallas import tpu as pltpu

_NB = 4
_K = 1024
_D = 128
_ROWS = 8 * 576
_TILE = 4608


def _vq_body(z_ref, cb_ref, c2_ref, codes_ref, inds_ref, comm_ref):
    j = pl.program_id(1)
    z = z_ref[...]
    cb = cb_ref[0]
    c2 = c2_ref[0, 0, :]
    dots = jnp.dot(z, cb.T, preferred_element_type=jnp.float32)
    z2 = jnp.sum(z * z, axis=1, keepdims=True)
    dist = z2 - 2.0 * dots + c2[None, :]
    m = jnp.min(dist, axis=1)
    lane_row = jax.lax.broadcasted_iota(jnp.int32, (1, _K), 1)
    lane_f = lane_row.astype(jnp.float32)
    idx_f = jnp.min(jnp.where(dist == m[:, None], lane_f,
                              jnp.float32(_K)), axis=1)
    idx = idx_f.astype(jnp.int32)
    onehot = (lane_row == idx[:, None]).astype(jnp.bfloat16)
    q = jnp.dot(onehot, cb.astype(jnp.bfloat16),
                preferred_element_type=jnp.float32)
    codes_ref[...] = q
    inds_ref[0, 0, :] = idx
    s = jnp.sum(m.reshape(_TILE // _D, _D), axis=0)

    @pl.when(j == 0)
    def _init():
        comm_ref[0, 0, :] = s

    @pl.when(j > 0)
    def _acc():
        comm_ref[0, 0, :] += s


@functools.partial(jax.jit)
def kernel(x, codebooks):
    b, n, D = x.shape
    xr = x.reshape(b * n, D)
    c2in = jnp.sum(codebooks * codebooks, axis=-1)[:, None, :]
    ntiles = _ROWS // _TILE
    codes, inds3, comm = pl.pallas_call(
        _vq_body,
        grid=(_NB, ntiles),
        in_specs=[
            pl.BlockSpec((_TILE, _D), lambda i, j: (j, i)),
            pl.BlockSpec((1, _K, _D), lambda i, j: (i, 0, 0)),
            pl.BlockSpec((1, 1, _K), lambda i, j: (i, 0, 0)),
        ],
        out_specs=[
            pl.BlockSpec((_TILE, _D), lambda i, j: (j, i)),
            pl.BlockSpec((1, 1, _TILE), lambda i, j: (i, 0, j)),
            pl.BlockSpec((1, 1, _D), lambda i, j: (i, 0, 0)),
        ],
        out_shape=[
            jax.ShapeDtypeStruct((_ROWS, _NB * _D), jnp.float32),
            jax.ShapeDtypeStruct((_NB, 1, _ROWS), jnp.int32),
            jax.ShapeDtypeStruct((_NB, 1, _D), jnp.float32),
        ],
    )(xr, codebooks, c2in)
    codes = codes.reshape(b, n, D)
    inds = inds3.reshape(_NB, b, n).transpose(1, 2, 0)
    commits = jnp.sum(comm[:, 0, :], axis=-1) / jnp.float32(_ROWS * _D)
    return (codes, inds, commits)
